# baseline (speedup 1.0000x reference)
import jax
import jax.numpy as jnp
from jax.experimental import pallas as pl

_N = 10000
_DIN = 256
_DOUT = 256

_BM1 = 2000
_BM2 = 400


def _support_body(xf_ref, w_ref, out_ref):
    out_ref[...] = jnp.dot(xf_ref[...], w_ref[...],
                           preferred_element_type=jnp.float32)


def _adj_body(adj_ref, s_ref, out_ref):
    out_ref[...] = jnp.dot(adj_ref[...], s_ref[...],
                           preferred_element_type=jnp.float32)


@jax.jit
def kernel(x, adj, W):
    xf = x.reshape(_N, _DIN)
    support = pl.pallas_call(
        _support_body,
        grid=(_N // _BM1,),
        in_specs=[
            pl.BlockSpec((_BM1, _DIN), lambda i: (i, 0)),
            pl.BlockSpec((_DIN, _DOUT), lambda i: (0, 0)),
        ],
        out_specs=pl.BlockSpec((_BM1, _DOUT), lambda i: (i, 0)),
        out_shape=jax.ShapeDtypeStruct((_N, _DOUT), jnp.float32),
    )(xf, W)
    out = pl.pallas_call(
        _adj_body,
        grid=(_N // _BM2,),
        in_specs=[
            pl.BlockSpec((_BM2, _N), lambda i: (i, 0)),
            pl.BlockSpec((_N, _DOUT), lambda i: (0, 0)),
        ],
        out_specs=pl.BlockSpec((_BM2, _DOUT), lambda i: (i, 0)),
        out_shape=jax.ShapeDtypeStruct((_N, _DOUT), jnp.float32),
    )(adj, support)
    return out


# speedup vs baseline: 1.0534x; 1.0534x over previous
import jax
import jax.numpy as jnp
from jax.experimental import pallas as pl
from jax.experimental.pallas import tpu as pltpu

_N = 10000
_DIN = 256
_DOUT = 256

_BM = 200


def _gcn_body(adj_ref, xf_ref, w_ref, out_ref, s_ref):
    @pl.when(pl.program_id(0) == 0)
    def _():
        s_ref[...] = jnp.dot(xf_ref[...], w_ref[...],
                             preferred_element_type=jnp.float32)

    out_ref[...] = jnp.dot(adj_ref[...], s_ref[...],
                           preferred_element_type=jnp.float32)


@jax.jit
def kernel(x, adj, W):
    xf = x.reshape(_N, _DIN)
    out = pl.pallas_call(
        _gcn_body,
        grid=(_N // _BM,),
        in_specs=[
            pl.BlockSpec((_BM, _N), lambda i: (i, 0)),
            pl.BlockSpec((_N, _DIN), lambda i: (0, 0)),
            pl.BlockSpec((_DIN, _DOUT), lambda i: (0, 0)),
        ],
        out_specs=pl.BlockSpec((_BM, _DOUT), lambda i: (i, 0)),
        out_shape=jax.ShapeDtypeStruct((_N, _DOUT), jnp.float32),
        scratch_shapes=[pltpu.VMEM((_N, _DOUT), jnp.float32)],
    )(adj, xf, W)
    return out


# speedup vs baseline: 1.0678x; 1.0137x over previous
import jax
import jax.numpy as jnp
from jax.experimental import pallas as pl
from jax.experimental.pallas import tpu as pltpu

_N = 10000
_DIN = 256
_DOUT = 256

_BM = 400


def _gcn_body(adj_ref, xf_ref, w_ref, out_ref, s_ref):
    @pl.when(pl.program_id(0) == 0)
    def _():
        s_ref[...] = jnp.dot(xf_ref[...], w_ref[...],
                             preferred_element_type=jnp.float32)

    out_ref[...] = jnp.dot(adj_ref[...], s_ref[...],
                           preferred_element_type=jnp.float32)


@jax.jit
def kernel(x, adj, W):
    xf = x.reshape(_N, _DIN)
    out = pl.pallas_call(
        _gcn_body,
        grid=(_N // _BM,),
        in_specs=[
            pl.BlockSpec((_BM, _N), lambda i: (i, 0)),
            pl.BlockSpec((_N, _DIN), lambda i: (0, 0)),
            pl.BlockSpec((_DIN, _DOUT), lambda i: (0, 0)),
        ],
        out_specs=pl.BlockSpec((_BM, _DOUT), lambda i: (i, 0)),
        out_shape=jax.ShapeDtypeStruct((_N, _DOUT), jnp.float32),
        scratch_shapes=[pltpu.VMEM((_N, _DOUT), jnp.float32)],
    )(adj, xf, W)
    return out
